# Initial kernel scaffold; baseline (speedup 1.0000x reference)
import jax
import jax.numpy as jnp
from jax.experimental import pallas as pl


def kernel(path, emb, W_ih0, W_hh0, b_ih0, b_hh0, W_ih1, W_hh1, b_ih1, b_hh1):
    raise NotImplementedError("write your pallas kernel here")


# speedup vs baseline: 3.9764x; 3.9764x over previous
import functools

import jax
import jax.numpy as jnp
from jax import lax
from jax.experimental import pallas as pl
from jax.experimental.pallas import tpu as pltpu
from jax.experimental.pallas import tpu_sc as plsc

V, D, H = 4316, 768, 768
B, L = 64, 50
G4 = 4 * H

NC, NS = 2, 16
NW = NC * NS
ROWS = B * L
ROWS_PAD = 3328
R_PER_W = ROWS_PAD // NW

CT = 5
CB = CT * B
NCHUNK = L // CT


def _gather_body(table_hbm, idx_hbm, out_hbm, idx_v, rows_v, sem):
    wid = lax.axis_index("s") * NC + lax.axis_index("c")
    base = wid * R_PER_W
    pltpu.sync_copy(idx_hbm.at[pl.ds(base, R_PER_W)], idx_v)
    pltpu.async_copy(table_hbm.at[idx_v], rows_v, sem).wait()
    pltpu.sync_copy(rows_v, out_hbm.at[pl.ds(base, R_PER_W)])


_sc_gather = pl.kernel(
    _gather_body,
    out_type=jax.ShapeDtypeStruct((ROWS_PAD, D), jnp.float32),
    mesh=plsc.VectorSubcoreMesh(
        core_axis_name="c", subcore_axis_name="s", num_cores=NC, num_subcores=NS
    ),
    scratch_types=[
        pltpu.VMEM((R_PER_W,), jnp.int32),
        pltpu.VMEM((R_PER_W, D), jnp.float32),
        pltpu.SemaphoreType.DMA,
    ],
)


def _lstm_body(
    pe_hbm, w0i, w0h, b0, w1i, w1h, b1,
    out_hbm, hn, cn,
    peb, xc, ys0c, outb, h0, c0, h1, c1, sem_in, sem_out,
):
    zero = jnp.zeros((B, H), jnp.float32)
    h0[...] = zero
    c0[...] = zero
    h1[...] = zero
    c1[...] = zero

    def lstm_step(gates, c_ref):
        i = jax.nn.sigmoid(gates[:, 0 * H:1 * H])
        f = jax.nn.sigmoid(gates[:, 1 * H:2 * H])
        g = jnp.tanh(gates[:, 2 * H:3 * H])
        o = jax.nn.sigmoid(gates[:, 3 * H:4 * H])
        c = f * c_ref[...] + i * g
        c_ref[...] = c
        return o * jnp.tanh(c)

    def chunk(k, _):
        cp_in = pltpu.make_async_copy(pe_hbm.at[pl.ds(k * CB, CB)], peb, sem_in)
        cp_in.start()
        cp_in.wait()

        xc[...] = jnp.dot(peb[...], w0i[...], preferred_element_type=jnp.float32) + b0[...]

        def step0(t, _):
            gates = xc[pl.ds(t * B, B), :] + jnp.dot(
                h0[...], w0h[...], preferred_element_type=jnp.float32
            )
            h = lstm_step(gates, c0)
            h0[...] = h
            ys0c[pl.ds(t * B, B), :] = h
            return 0

        lax.fori_loop(0, CT, step0, 0)

        xc[...] = jnp.dot(ys0c[...], w1i[...], preferred_element_type=jnp.float32) + b1[...]

        def step1(t, _):
            gates = xc[pl.ds(t * B, B), :] + jnp.dot(
                h1[...], w1h[...], preferred_element_type=jnp.float32
            )
            h = lstm_step(gates, c1)
            h1[...] = h
            outb[pl.ds(t * B, B), :] = h
            return 0

        lax.fori_loop(0, CT, step1, 0)

        cp_out = pltpu.make_async_copy(outb, out_hbm.at[pl.ds(k * CB, CB)], sem_out)
        cp_out.start()
        cp_out.wait()
        return 0

    lax.fori_loop(0, NCHUNK, chunk, 0)

    hn[0, :, :] = h0[...]
    hn[1, :, :] = h1[...]
    cn[0, :, :] = c0[...]
    cn[1, :, :] = c1[...]


_lstm = pl.pallas_call(
    _lstm_body,
    out_shape=[
        jax.ShapeDtypeStruct((ROWS, H), jnp.float32),
        jax.ShapeDtypeStruct((2, B, H), jnp.float32),
        jax.ShapeDtypeStruct((2, B, H), jnp.float32),
    ],
    in_specs=[
        pl.BlockSpec(memory_space=pl.ANY),
        pl.BlockSpec(memory_space=pltpu.MemorySpace.VMEM),
        pl.BlockSpec(memory_space=pltpu.MemorySpace.VMEM),
        pl.BlockSpec(memory_space=pltpu.MemorySpace.VMEM),
        pl.BlockSpec(memory_space=pltpu.MemorySpace.VMEM),
        pl.BlockSpec(memory_space=pltpu.MemorySpace.VMEM),
        pl.BlockSpec(memory_space=pltpu.MemorySpace.VMEM),
    ],
    out_specs=[
        pl.BlockSpec(memory_space=pl.ANY),
        pl.BlockSpec(memory_space=pltpu.MemorySpace.VMEM),
        pl.BlockSpec(memory_space=pltpu.MemorySpace.VMEM),
    ],
    scratch_shapes=[
        pltpu.VMEM((CB, D), jnp.float32),
        pltpu.VMEM((CB, G4), jnp.float32),
        pltpu.VMEM((CB, H), jnp.float32),
        pltpu.VMEM((CB, H), jnp.float32),
        pltpu.VMEM((B, H), jnp.float32),
        pltpu.VMEM((B, H), jnp.float32),
        pltpu.VMEM((B, H), jnp.float32),
        pltpu.VMEM((B, H), jnp.float32),
        pltpu.SemaphoreType.DMA,
        pltpu.SemaphoreType.DMA,
    ],
)


@jax.jit
def kernel(path, emb, W_ih0, W_hh0, b_ih0, b_hh0, W_ih1, W_hh1, b_ih1, b_hh1):
    idx = path.astype(jnp.int32).T.reshape(-1)
    idx_pad = jnp.concatenate([idx, jnp.zeros((ROWS_PAD - ROWS,), jnp.int32)])
    pe_t = _sc_gather(emb, idx_pad)[:ROWS]
    pe = pe_t.reshape(L, B, D).transpose(1, 0, 2)

    out_t, hn, cn = _lstm(
        pe_t,
        W_ih0.T, W_hh0.T, (b_ih0 + b_hh0).reshape(1, G4),
        W_ih1.T, W_hh1.T, (b_ih1 + b_hh1).reshape(1, G4),
    )
    outputs = out_t.reshape(L, B, H).transpose(1, 0, 2)
    return outputs, (hn, cn), pe
